# baseline (speedup 1.0000x reference)
import functools

import jax
import jax.numpy as jnp
from jax import lax
from jax.experimental import pallas as pl
from jax.experimental.pallas import tpu as pltpu
from jax.experimental.pallas import tpu_sc as plsc

NENTITY = 1000000
NRELATION = 1000
ED = 32
GAMMA = 12.0
B = 1024
NEG = 256

NC = 2
NS = 16
NW = NC * NS
BPW = B // NW
CHUNK = 128
NCHUNK = B * NEG // (NW * CHUNK)
NBUF = 4
ROW = 128


def _rsqrt(s, iters):
    i = lax.bitcast_convert_type(s, jnp.int32)
    i = jnp.int32(0x5F3759DF) - lax.shift_right_logical(i, 1)
    y = lax.bitcast_convert_type(i, jnp.float32)
    xh = 0.5 * s
    for _ in range(iters):
        y = y * (1.5 - xh * y * y)
    return y


def _build_sc_kernel():
    mesh = plsc.VectorSubcoreMesh(core_axis_name="c", subcore_axis_name="s")

    @functools.partial(
        pl.kernel,
        mesh=mesh,
        out_type=jax.ShapeDtypeStruct((B * NEG,), jnp.float32),
        compiler_params=pltpu.CompilerParams(
            needs_layout_passes=False, use_tc_tiling_on_sc=False),
        scratch_types=[
            pltpu.VMEM((BPW,), jnp.int32),
            pltpu.VMEM((BPW,), jnp.int32),
            pltpu.VMEM((BPW,), jnp.int32),
            pltpu.VMEM((BPW, ROW), jnp.float32),
            pltpu.VMEM((BPW, 4 * ED), jnp.float32),
            pltpu.VMEM((BPW * ED,), jnp.float32),
            pltpu.VMEM((BPW * ED,), jnp.float32),
            pltpu.VMEM((ED * 16,), jnp.int32),
            pltpu.VMEM((ED * 16,), jnp.float32),
            pltpu.VMEM((ED * 16,), jnp.float32),
            pltpu.VMEM((NCHUNK, CHUNK), jnp.int32),
            pltpu.VMEM((NCHUNK, CHUNK), jnp.int32),
            pltpu.VMEM((NBUF, CHUNK, ROW), jnp.float32),
            pltpu.VMEM((NCHUNK * CHUNK,), jnp.float32),
            pltpu.SemaphoreType.DMA,
            pltpu.SemaphoreType.DMA,
            pltpu.SemaphoreType.DMA,
            pltpu.SemaphoreType.DMA,
            pltpu.SemaphoreType.DMA,
            pltpu.SemaphoreType.DMA,
        ],
    )
    def kern(ent_hbm, rel_hbm, hrow_hbm, hpar_hbm, rid_hbm, trow_hbm,
             tpar_hbm, out_hbm,
             hrow, hpar, ridx, headbuf, relbuf, hx, hy, colrot, hxrot, hyrot,
             trow, tpar, tailbuf, scores, semh, semr, sem0, sem1, sem2, sem3):
        wid = lax.axis_index("s") * NC + lax.axis_index("c")
        base_b = wid * BPW
        sems = [sem0, sem1, sem2, sem3]

        pltpu.sync_copy(hrow_hbm.at[pl.ds(base_b, BPW)], hrow)
        pltpu.sync_copy(hpar_hbm.at[pl.ds(base_b, BPW)], hpar)
        pltpu.sync_copy(rid_hbm.at[pl.ds(base_b, BPW)], ridx)
        pltpu.sync_copy(trow_hbm.at[pl.ds(wid * NCHUNK, NCHUNK)], trow)
        pltpu.sync_copy(tpar_hbm.at[pl.ds(wid * NCHUNK, NCHUNK)], tpar)
        head_cp = pltpu.make_async_copy(ent_hbm.at[hrow], headbuf, semh)
        head_cp.start()
        rel_cp = pltpu.make_async_copy(rel_hbm.at[ridx], relbuf, semr)
        rel_cp.start()
        for j in range(NBUF):
            pltpu.make_async_copy(
                ent_hbm.at[trow.at[j]], tailbuf.at[j], sems[j]).start()
        head_cp.wait()
        rel_cp.wait()

        iot = lax.iota(jnp.int32, 16)
        for kk in range(ED):
            colrot[pl.ds(kk * 16, 16)] = ((iot + kk) & (ED - 1)) * 2

        def prep_b(b, carry):
            rb = jnp.full((16,), b, jnp.int32)
            parb = plsc.load_gather(hpar, [rb])

            def norm2(x, y):
                inv = _rsqrt(x * x + y * y, 3)
                return x * inv, y * inv

            def refl(phx, phy, rx, ry):
                d2 = 2.0 * (rx * phx + ry * phy)
                return phx - d2 * rx, phy - d2 * ry

            for half in range(2):
                rco = 64 * half
                hc = parb + iot * 2 + 32 * half
                phx = plsc.load_gather(headbuf, [rb, hc])
                phy = plsc.load_gather(headbuf, [rb, hc + 1])
                r0x = plsc.load_gather(relbuf, [rb, iot * 4 + rco])
                r0y = plsc.load_gather(relbuf, [rb, iot * 4 + rco + 1])
                r1x = plsc.load_gather(relbuf, [rb, iot * 4 + rco + 2])
                r1y = plsc.load_gather(relbuf, [rb, iot * 4 + rco + 3])
                r0x, r0y = norm2(r0x, r0y)
                r1x, r1y = norm2(r1x, r1y)
                phx, phy = refl(phx, phy, r1x, r1y)
                phx, phy = refl(phx, phy, r0x, r0y)
                hx[pl.ds(b * ED + 16 * half, 16)] = phx
                hy[pl.ds(b * ED + 16 * half, 16)] = phy
            return carry

        lax.fori_loop(0, BPW, prep_b, 0)

        def chunk_group(cc, carry):
            for j in range(NBUF):
                c = cc * NBUF + j
                buf = tailbuf.at[j]
                pltpu.make_async_copy(
                    ent_hbm.at[trow.at[c]], buf, sems[j]).wait()
                kbase = lax.shift_right_logical(c, 1) * ED

                @pl.when((c & 1) == 0)
                def _():
                    def rot_k(kk, carry2):
                        src = lax.shift_right_logical(
                            colrot[pl.ds(kk * 16, 16)], 1) + kbase
                        hxrot[pl.ds(kk * 16, 16)] = plsc.load_gather(hx, [src])
                        hyrot[pl.ds(kk * 16, 16)] = plsc.load_gather(hy, [src])
                        return carry2

                    lax.fori_loop(0, ED, rot_k, 0)

                def g_body(g, carry2):
                    rowv = iot + g * 16
                    parv = tpar[c, pl.ds(g * 16, 16)]
                    acc = jnp.zeros((16,), jnp.float32)
                    for kk in range(ED):
                        colx = parv + colrot[pl.ds(kk * 16, 16)]
                        gx = plsc.load_gather(buf, [rowv, colx])
                        gy = plsc.load_gather(buf, [rowv, colx + 1])
                        dx = gx - hxrot[pl.ds(kk * 16, 16)]
                        dy = gy - hyrot[pl.ds(kk * 16, 16)]
                        s = dx * dx + dy * dy
                        acc = acc + s * _rsqrt(s, 2)
                    scores[pl.ds(c * CHUNK + g * 16, 16)] = GAMMA - acc
                    return carry2

                lax.fori_loop(0, 8, g_body, 0)
                nc = c + NBUF

                @pl.when(nc < NCHUNK)
                def _():
                    pltpu.make_async_copy(
                        ent_hbm.at[trow.at[nc]], buf, sems[j]).start()
            return carry

        lax.fori_loop(0, NCHUNK // NBUF, chunk_group, 0)
        pltpu.sync_copy(
            scores, out_hbm.at[pl.ds(wid * NCHUNK * CHUNK, NCHUNK * CHUNK)])

    return kern


_SC_KERNEL = _build_sc_kernel()


def kernel(head_part, tail_part, entity_embedding, relation_embedding):
    ent = entity_embedding.reshape(NENTITY // 2, ROW)
    rel = relation_embedding.reshape(NRELATION, 4 * ED)
    hid = head_part[:, 0]
    hrow = hid >> 1
    hpar = (hid & 1) * 64
    rid = head_part[:, 1]
    trow = (tail_part >> 1).reshape(B * NEG // CHUNK, CHUNK)
    tpar = ((tail_part & 1) * 64).reshape(B * NEG // CHUNK, CHUNK)
    out = _SC_KERNEL(ent, rel, hrow, hpar, rid, trow, tpar)
    return out.reshape(B, NEG)


# speedup vs baseline: 38.5104x; 38.5104x over previous
import functools

import jax
import jax.numpy as jnp
from jax import lax
from jax.experimental import pallas as pl
from jax.experimental.pallas import tpu as pltpu
from jax.experimental.pallas import tpu_sc as plsc

NENTITY = 1000000
NRELATION = 1000
ED = 32
GAMMA = 12.0
B = 1024
NEG = 256

NC = 2
NS = 16
NW = NC * NS
BPW = B // NW
CHUNK = 128
NCHUNK = B * NEG // (NW * CHUNK)
NBUF = 4
ROW = 128


def _rsqrt(s, iters):
    i = lax.bitcast_convert_type(s, jnp.int32)
    i = jnp.int32(0x5F3759DF) - lax.shift_right_logical(i, 1)
    y = lax.bitcast_convert_type(i, jnp.float32)
    xh = 0.5 * s
    for _ in range(iters):
        y = y * (1.5 - xh * y * y)
    return y


def _unpack_pairs(words):
    bf = plsc.bitcast(words, jnp.bfloat16)
    return plsc.unpack(bf, format=plsc.PackFormat.INTERLEAVED)


def _build_sc_kernel():
    mesh = plsc.VectorSubcoreMesh(core_axis_name="c", subcore_axis_name="s")

    @functools.partial(
        pl.kernel,
        mesh=mesh,
        out_type=jax.ShapeDtypeStruct((B * NEG,), jnp.float32),
        compiler_params=pltpu.CompilerParams(
            needs_layout_passes=False, use_tc_tiling_on_sc=False),
        scratch_types=[
            pltpu.VMEM((BPW,), jnp.int32),
            pltpu.VMEM((BPW,), jnp.int32),
            pltpu.VMEM((BPW,), jnp.int32),
            pltpu.VMEM((BPW, ROW), jnp.int32),
            pltpu.VMEM((BPW, 4 * ED), jnp.float32),
            pltpu.VMEM((BPW * ED,), jnp.float32),
            pltpu.VMEM((BPW * ED,), jnp.float32),
            pltpu.VMEM((ED * 16,), jnp.int32),
            pltpu.VMEM((ED * 16,), jnp.float32),
            pltpu.VMEM((ED * 16,), jnp.float32),
            pltpu.VMEM((NCHUNK, CHUNK), jnp.int32),
            pltpu.VMEM((NCHUNK, CHUNK), jnp.int32),
            pltpu.VMEM((NBUF, CHUNK, ROW), jnp.int32),
            pltpu.VMEM((NCHUNK * CHUNK,), jnp.float32),
            pltpu.SemaphoreType.DMA,
            pltpu.SemaphoreType.DMA,
            pltpu.SemaphoreType.DMA,
            pltpu.SemaphoreType.DMA,
            pltpu.SemaphoreType.DMA,
            pltpu.SemaphoreType.DMA,
        ],
    )
    def kern(ent_hbm, rel_hbm, hrow_hbm, hsub_hbm, rid_hbm, trow_hbm,
             tsub_hbm, out_hbm,
             hrow, hsub, ridx, headbuf, relbuf, hx, hy, colrot, hxrot, hyrot,
             trow, tsub, tailbuf, scores, semh, semr, sem0, sem1, sem2, sem3):
        wid = lax.axis_index("s") * NC + lax.axis_index("c")
        base_b = wid * BPW
        sems = [sem0, sem1, sem2, sem3]

        pltpu.sync_copy(hrow_hbm.at[pl.ds(base_b, BPW)], hrow)
        pltpu.sync_copy(hsub_hbm.at[pl.ds(base_b, BPW)], hsub)
        pltpu.sync_copy(rid_hbm.at[pl.ds(base_b, BPW)], ridx)
        pltpu.sync_copy(trow_hbm.at[pl.ds(wid * NCHUNK, NCHUNK)], trow)
        pltpu.sync_copy(tsub_hbm.at[pl.ds(wid * NCHUNK, NCHUNK)], tsub)
        head_cp = pltpu.make_async_copy(ent_hbm.at[hrow], headbuf, semh)
        head_cp.start()
        rel_cp = pltpu.make_async_copy(rel_hbm.at[ridx], relbuf, semr)
        rel_cp.start()
        for j in range(NBUF):
            pltpu.make_async_copy(
                ent_hbm.at[trow.at[j]], tailbuf.at[j], sems[j]).start()
        head_cp.wait()
        rel_cp.wait()

        iot = lax.iota(jnp.int32, 16)
        for kk in range(ED):
            colrot[pl.ds(kk * 16, 16)] = (iot + kk) & (ED - 1)

        def prep_b(b, carry):
            rb = jnp.full((16,), b, jnp.int32)
            subb = plsc.load_gather(hsub, [rb])

            def norm2(x, y):
                inv = _rsqrt(x * x + y * y, 3)
                return x * inv, y * inv

            def refl(phx, phy, rx, ry):
                d2 = 2.0 * (rx * phx + ry * phy)
                return phx - d2 * rx, phy - d2 * ry

            for half in range(2):
                rco = 64 * half
                phx, phy = _unpack_pairs(plsc.load_gather(
                    headbuf, [rb, subb + iot + 16 * half]))
                r0x = plsc.load_gather(relbuf, [rb, iot * 4 + rco])
                r0y = plsc.load_gather(relbuf, [rb, iot * 4 + rco + 1])
                r1x = plsc.load_gather(relbuf, [rb, iot * 4 + rco + 2])
                r1y = plsc.load_gather(relbuf, [rb, iot * 4 + rco + 3])
                r0x, r0y = norm2(r0x, r0y)
                r1x, r1y = norm2(r1x, r1y)
                phx, phy = refl(phx, phy, r1x, r1y)
                phx, phy = refl(phx, phy, r0x, r0y)
                hx[pl.ds(b * ED + 16 * half, 16)] = phx
                hy[pl.ds(b * ED + 16 * half, 16)] = phy
            return carry

        lax.fori_loop(0, BPW, prep_b, 0)

        def chunk_group(cc, carry):
            for j in range(NBUF):
                c = cc * NBUF + j
                buf = tailbuf.at[j]
                pltpu.make_async_copy(
                    ent_hbm.at[trow.at[c]], buf, sems[j]).wait()
                kbase = lax.shift_right_logical(c, 1) * ED

                @pl.when((c & 1) == 0)
                def _():
                    def rot_k(kk, carry2):
                        src = colrot[pl.ds(kk * 16, 16)] + kbase
                        hxrot[pl.ds(kk * 16, 16)] = plsc.load_gather(hx, [src])
                        hyrot[pl.ds(kk * 16, 16)] = plsc.load_gather(hy, [src])
                        return carry2

                    lax.fori_loop(0, ED, rot_k, 0)

                def g_body(g, carry2):
                    rowv0 = iot + g * 32
                    rowv1 = rowv0 + 16
                    subv0 = tsub[c, pl.ds(g * 32, 16)]
                    subv1 = tsub[c, pl.ds(g * 32 + 16, 16)]
                    accs = [jnp.zeros((16,), jnp.float32) for _ in range(4)]
                    crr = iot
                    for kk in range(ED):
                        hxk = hxrot[pl.ds(kk * 16, 16)]
                        hyk = hyrot[pl.ds(kk * 16, 16)]
                        col0 = subv0 + crr
                        col1 = subv1 + crr
                        crr = (crr + 1) & (ED - 1)
                        gx0, gy0 = _unpack_pairs(
                            plsc.load_gather(buf, [rowv0, col0]))
                        gx1, gy1 = _unpack_pairs(
                            plsc.load_gather(buf, [rowv1, col1]))
                        dx0 = gx0 - hxk
                        dy0 = gy0 - hyk
                        dx1 = gx1 - hxk
                        dy1 = gy1 - hyk
                        s0 = dx0 * dx0 + dy0 * dy0
                        s1 = dx1 * dx1 + dy1 * dy1
                        k2 = kk & 1
                        accs[k2] = accs[k2] + s0 * _rsqrt(s0, 1)
                        accs[2 + k2] = accs[2 + k2] + s1 * _rsqrt(s1, 1)
                    base = c * CHUNK + g * 32
                    scores[pl.ds(base, 16)] = GAMMA - (accs[0] + accs[1])
                    scores[pl.ds(base + 16, 16)] = GAMMA - (accs[2] + accs[3])
                    return carry2

                lax.fori_loop(0, 4, g_body, 0)
                nc = c + NBUF

                @pl.when(nc < NCHUNK)
                def _():
                    pltpu.make_async_copy(
                        ent_hbm.at[trow.at[nc]], buf, sems[j]).start()
            return carry

        lax.fori_loop(0, NCHUNK // NBUF, chunk_group, 0)
        pltpu.sync_copy(
            scores, out_hbm.at[pl.ds(wid * NCHUNK * CHUNK, NCHUNK * CHUNK)])

    return kern


_SC_KERNEL = _build_sc_kernel()


def kernel(head_part, tail_part, entity_embedding, relation_embedding):
    ent = lax.bitcast_convert_type(
        entity_embedding.astype(jnp.bfloat16), jnp.int32)
    ent = ent.reshape(NENTITY // 4, ROW)
    rel = relation_embedding.reshape(NRELATION, 4 * ED)
    hid = head_part[:, 0]
    hrow = hid >> 2
    hsub = (hid & 3) * ED
    rid = head_part[:, 1]
    trow = (tail_part >> 2).reshape(B * NEG // CHUNK, CHUNK)
    tsub = ((tail_part & 3) * ED).reshape(B * NEG // CHUNK, CHUNK)
    out = _SC_KERNEL(ent, rel, hrow, hsub, rid, trow, tsub)
    return out.reshape(B, NEG)
